# baseline (speedup 1.0000x reference)
import functools

import jax
import jax.numpy as jnp
from jax.experimental import pallas as pl
from jax.experimental.pallas import tpu as pltpu
from jax.experimental.pallas import tpu_sc as plsc

D_MODEL_DIM = 64
WINDOW = 128
SCALE = 8.0


@functools.partial(jax.jit, static_argnames=("num_indices",))
def _gather_scaled(table, idx_flat, num_indices):
    mesh = plsc.VectorSubcoreMesh(core_axis_name="core", subcore_axis_name="subcore")

    @functools.partial(
        pl.kernel,
        out_type=jax.ShapeDtypeStruct((num_indices, D_MODEL_DIM), jnp.float32),
        mesh=mesh,
        compiler_params=pltpu.CompilerParams(use_tc_tiling_on_sc=False),
    )
    def k(table_hbm, idx_hbm, out_hbm):
        def body(i_vmem, o_vmem):
            pltpu.sync_copy(table_hbm.at[i_vmem.at[0]], o_vmem)

            @pl.loop(0, WINDOW, step=4)
            def _(r):
                for rr in range(4):
                    for c in range(0, D_MODEL_DIM, 16):
                        slc = (pl.ds(r + rr, 1), pl.ds(c, 16))
                        o_vmem.at[slc][...] = o_vmem.at[slc][...] * SCALE

        pltpu.emit_pipeline(
            body,
            grid=(num_indices // WINDOW,),
            in_specs=[pl.BlockSpec((1, WINDOW), index_map=lambda i: (0, i))],
            out_specs=[pl.BlockSpec((WINDOW, D_MODEL_DIM), index_map=lambda i: (i, 0))],
            core_axis_name=("core", "subcore"),
            dimension_semantics=(pltpu.PARALLEL,),
        )(idx_hbm, out_hbm)

    return k(table, idx_flat)


def kernel(x, table):
    b = x.shape[0] * x.shape[1]
    idx_flat = x.reshape(1, b).astype(jnp.int32)
    out = _gather_scaled(table, idx_flat, b)
    return out.reshape(x.shape + (D_MODEL_DIM,))
